# baseline (speedup 1.0000x reference)
import functools

import jax
import jax.numpy as jnp
from jax import lax
from jax.experimental import pallas as pl
from jax.experimental.pallas import tpu as pltpu
from jax.experimental.pallas import tpu_sc as plsc

N_M = 800
N_D = 800
E = 50000
SLOPE = 0.2

NC = 2
NS = 16
CHUNK = 3136
E_PAD = NS * CHUNK
SUB = 112
NSUB = CHUNK // SUB
A_N = N_M * N_D
A_SIZE = 641024
ZCHUNK = A_SIZE // NS
OCHUNK = A_N // NS


def _leaky(x):
    return jnp.where(x >= 0, x, SLOPE * x)


def _adj_body(src0_hbm, dst0_hbm, src1_hbm, dst1_hbm, out_hbm,
              src_v, dst_v, idx_v, ones_v, stripe_v, a_sh, sem):
    c = lax.axis_index("c")
    s = lax.axis_index("s")

    def zbody(i, _):
        for u in range(8):
            stripe_v[pl.ds((i * 8 + u) * 16, 16)] = jnp.zeros((16,), jnp.float32)
        return 0
    lax.fori_loop(0, ZCHUNK // 128, zbody, 0)
    zdma = pltpu.async_copy(stripe_v, a_sh.at[pl.ds(s * ZCHUNK, ZCHUNK)], sem)

    base = s * CHUNK

    @pl.when(c == 0)
    def _():
        pltpu.sync_copy(src0_hbm.at[pl.ds(base, CHUNK)], src_v)
        pltpu.sync_copy(dst0_hbm.at[pl.ds(base, CHUNK)], dst_v)

    @pl.when(c == 1)
    def _():
        pltpu.sync_copy(src1_hbm.at[pl.ds(base, CHUNK)], src_v)
        pltpu.sync_copy(dst1_hbm.at[pl.ds(base, CHUNK)], dst_v)
    for u in range(SUB // 16):
        ones_v[pl.ds(u * 16, 16)] = jnp.ones((16,), jnp.float32)

    zdma.wait()
    plsc.subcore_barrier()

    descs = []
    for t in range(NSUB):
        for u in range(SUB // 16):
            off = t * SUB + u * 16
            sv = src_v[pl.ds(off, 16)]
            dv = dst_v[pl.ds(off, 16)]
            idx_v[t, pl.ds(u * 16, 16)] = dv * N_D + sv
        descs.append(pltpu.async_copy(ones_v, a_sh.at[idx_v.at[t]], sem,
                                      add=True))
    for d in descs:
        d.wait()

    plsc.subcore_barrier()

    pltpu.sync_copy(a_sh.at[pl.ds(s * OCHUNK, OCHUNK)],
                    stripe_v.at[pl.ds(0, OCHUNK)])
    pltpu.sync_copy(stripe_v.at[pl.ds(0, OCHUNK)],
                    out_hbm.at[pl.ds(c * A_N + s * OCHUNK, OCHUNK)])


def _build_adj(src0, dst0, src1, dst1):
    mesh = plsc.VectorSubcoreMesh(core_axis_name="c", subcore_axis_name="s")
    fn = pl.kernel(
        _adj_body,
        out_type=jax.ShapeDtypeStruct((2 * A_N,), jnp.float32),
        mesh=mesh,
        scratch_types=[
            pltpu.VMEM((CHUNK,), jnp.int32),
            pltpu.VMEM((CHUNK,), jnp.int32),
            pltpu.VMEM((NSUB, SUB), jnp.int32),
            pltpu.VMEM((SUB,), jnp.float32),
            pltpu.VMEM((ZCHUNK,), jnp.float32),
            pltpu.VMEM_SHARED((A_SIZE,), jnp.float32),
            pltpu.SemaphoreType.DMA,
        ],
    )
    return fn(src0, dst0, src1, dst1)


KB = 1920
KSTEPS = 96000 // KB


def _emb_body(emb_ref, wu_ref, we_ref, bu_ref, be_ref, u_out, y_out,
              acc_u, acc_y):
    k = pl.program_id(0)

    @pl.when(k == 0)
    def _():
        acc_u[...] = jnp.zeros_like(acc_u)
        acc_y[...] = jnp.zeros_like(acc_y)

    e = emb_ref[...].astype(jnp.bfloat16)
    dn = (((1,), (1,)), ((), ()))
    acc_u[...] += lax.dot_general(e, wu_ref[...].astype(jnp.bfloat16), dn,
                                  preferred_element_type=jnp.float32)
    acc_y[...] += lax.dot_general(e, we_ref[...].astype(jnp.bfloat16), dn,
                                  preferred_element_type=jnp.float32)

    @pl.when(k == KSTEPS - 1)
    def _():
        u_out[...] = _leaky(acc_u[...] + bu_ref[...][None, :]).astype(jnp.bfloat16)
        y_out[...] = acc_y[...] + be_ref[...][None, :]


def _emb_proj(emb, wu, bu, we, be):
    return pl.pallas_call(
        _emb_body,
        grid=(KSTEPS,),
        in_specs=[
            pl.BlockSpec((N_M, KB), lambda k: (0, k)),
            pl.BlockSpec((1024, KB), lambda k: (0, k)),
            pl.BlockSpec((128, KB), lambda k: (0, k)),
            pl.BlockSpec((1024,), lambda k: (0,)),
            pl.BlockSpec((128,), lambda k: (0,)),
        ],
        out_specs=[
            pl.BlockSpec((N_M, 1024), lambda k: (0, 0)),
            pl.BlockSpec((N_M, 128), lambda k: (0, 0)),
        ],
        out_shape=[
            jax.ShapeDtypeStruct((N_M, 1024), jnp.bfloat16),
            jax.ShapeDtypeStruct((N_M, 128), jnp.float32),
        ],
        scratch_shapes=[
            pltpu.VMEM((N_M, 1024), jnp.float32),
            pltpu.VMEM((N_M, 128), jnp.float32),
        ],
    )(emb, wu, we, bu, be)


def _dense_body(msim, mass, dsim, dass, a_m2d, a_d2m, u_emb, y128, scal,
                w_msim_l, b_msim_l, w_mass_l, b_mass_l,
                w_dsim_l, b_dsim_l, w_dass_l, b_dass_l,
                w_umsim, b_umsim, w_umass, b_umass,
                w_udsim, b_udsim, w_udass, b_udass,
                w_emb_u, b_emb_u, w_msim_u, b_msim_u, w_mass_u, b_mass_u,
                w_dsim_u, b_dsim_u, w_dass_u, b_dass_u,
                s0l_w, s0l_b, s0r_w, s0r_b, s1l_w, s1l_b, s1r_w, s1r_b,
                s2l_w, s2l_b, s2r_w, s2r_b,
                t0l_w, t0l_b, t0r_w, t0r_b, t1l_w, t1l_b, t1r_w, t1r_b,
                t2l_w, t2l_b, t2r_w, t2r_b,
                out_m, out_d,
                msim_v, mass_v, dsim_v, dass_v, wum_v, wua_v, wuds_v, wuda_v,
                uemb_v, y128_v, am_v, ad_v, sems):
    dn = (((1,), (1,)), ((), ()))

    def mm(x, w):
        return lax.dot_general(x.astype(jnp.bfloat16), w[...].astype(jnp.bfloat16),
                               dn, preferred_element_type=jnp.float32)

    def app(x, w, b):
        return mm(x, w) + b[...][None, :]

    big = [(y128, y128_v), (msim, msim_v), (mass, mass_v),
           (dsim, dsim_v), (dass, dass_v),
           (w_umsim, wum_v), (w_umass, wua_v),
           (w_udsim, wuds_v), (w_udass, wuda_v),
           (u_emb, uemb_v),
           (a_d2m, am_v), (a_m2d, ad_v)]
    descs = [pltpu.async_copy(s, d, sems.at[i]) for i, (s, d) in enumerate(big)]

    e = jnp.exp(scal[...])
    def sm(base, n, i):
        tot = e[base]
        for j in range(1, n):
            tot = tot + e[base + j]
        return e[base + i] / tot
    w_m0, w_m1, w_m2 = sm(0, 3, 0), sm(0, 3, 1), sm(0, 3, 2)
    w_d0, w_d1 = sm(3, 2, 0), sm(3, 2, 1)
    w_mu0, w_mu1, w_mu2 = sm(5, 3, 0), sm(5, 3, 1), sm(5, 3, 2)
    w_du0, w_du1 = sm(8, 2, 0), sm(8, 2, 1)
    ag_m0, ag_m1 = sm(10, 2, 0), sm(10, 2, 1)
    ag_d0, ag_d1 = sm(12, 2, 0), sm(12, 2, 1)

    descs[0].wait()
    descs[1].wait()
    descs[2].wait()
    ms = msim_v[...]
    ma = mass_v[...]
    x_m = y128_v[...] * w_m0 + app(ms, w_msim_l, b_msim_l) * w_m1 \
        + app(ma, w_mass_l, b_mass_l) * w_m2

    descs[3].wait()
    descs[4].wait()
    ds_ = dsim_v[...]
    da = dass_v[...]
    x_d = app(ds_, w_dsim_l, b_dsim_l) * w_d0 + app(da, w_dass_l, b_dass_l) * w_d1

    descs[5].wait()
    u_msim = _leaky(app(ms, wum_v, b_umsim))
    descs[6].wait()
    u_mass = _leaky(app(ma, wua_v, b_umass))
    descs[7].wait()
    u_dsim = _leaky(app(ds_, wuds_v, b_udsim))
    descs[8].wait()
    u_dass = _leaky(app(da, wuda_v, b_udass))

    descs[9].wait()
    xm = app(uemb_v[...], w_emb_u, b_emb_u) * w_mu0 \
        + app(u_msim, w_msim_u, b_msim_u) * w_mu1 \
        + app(u_mass, w_mass_u, b_mass_u) * w_mu2
    xd = app(u_dsim, w_dsim_u, b_dsim_u) * w_du0 \
        + app(u_dass, w_dass_u, b_dass_u) * w_du1

    descs[10].wait()
    descs[11].wait()
    am = am_v[...]
    ad = ad_v[...]
    inv_m = 1.0 / jnp.maximum(jnp.sum(am, axis=1, keepdims=True), 1.0)
    inv_d = 1.0 / jnp.maximum(jnp.sum(ad, axis=1, keepdims=True), 1.0)

    def mean_m(x):
        return jnp.dot(am.astype(jnp.bfloat16), x.astype(jnp.bfloat16),
                       preferred_element_type=jnp.float32) * inv_m

    def mean_d(x):
        return jnp.dot(ad.astype(jnp.bfloat16), x.astype(jnp.bfloat16),
                       preferred_element_type=jnp.float32) * inv_d

    nm = app(mean_m(xd), s0l_w, s0l_b) + app(xm, s0r_w, s0r_b)
    nd = app(mean_d(xm), t0l_w, t0l_b) + app(xd, t0r_w, t0r_b)
    xm, xd = _leaky(nm), _leaky(nd)
    nm = app(mean_m(xd), s1l_w, s1l_b) + app(xm, s1r_w, s1r_b)
    nd = app(mean_d(xm), t1l_w, t1l_b) + app(xd, t1r_w, t1r_b)
    xm, xd = _leaky(nm), _leaky(nd)
    nm = app(mean_m(xd), s2l_w, s2l_b) + app(xm, s2r_w, s2r_b)
    nd = app(mean_d(xm), t2l_w, t2l_b) + app(xd, t2r_w, t2r_b)

    out_m[...] = x_m * ag_m0 + nm * ag_m1
    out_d[...] = x_d * ag_d0 + nd * ag_d1


_BIG_W_IDX = {8, 10, 12, 14}


def _dense(msim, mass, dsim, dass, a_m2d, a_d2m, u_emb, y128, scal, weights):
    in_specs = ([pl.BlockSpec(memory_space=pl.ANY)] * 8
                + [pl.BlockSpec((14,), lambda: (0,))]
                + [pl.BlockSpec(memory_space=pl.ANY) if i in _BIG_W_IDX
                   else pl.BlockSpec(w.shape, lambda nd=w.ndim: (0,) * nd)
                   for i, w in enumerate(weights)])
    return pl.pallas_call(
        _dense_body,
        in_specs=in_specs,
        out_specs=[
            pl.BlockSpec((N_M, 128), lambda: (0, 0)),
            pl.BlockSpec((N_D, 128), lambda: (0, 0)),
        ],
        out_shape=[
            jax.ShapeDtypeStruct((N_M, 128), jnp.float32),
            jax.ShapeDtypeStruct((N_D, 128), jnp.float32),
        ],
        scratch_shapes=[
            pltpu.VMEM((N_M, N_M), jnp.float32),
            pltpu.VMEM((N_M, N_D), jnp.float32),
            pltpu.VMEM((N_D, N_D), jnp.float32),
            pltpu.VMEM((N_D, N_M), jnp.float32),
            pltpu.VMEM((N_M, N_M), jnp.float32),
            pltpu.VMEM((N_D, N_D), jnp.float32),
            pltpu.VMEM((N_D, N_D), jnp.float32),
            pltpu.VMEM((N_M, N_M), jnp.float32),
            pltpu.VMEM((N_M, 1024), jnp.bfloat16),
            pltpu.VMEM((N_M, 128), jnp.float32),
            pltpu.VMEM((N_M, N_D), jnp.float32),
            pltpu.VMEM((N_D, N_M), jnp.float32),
            pltpu.SemaphoreType.DMA((12,)),
        ],
    )(msim, mass, dsim, dass, a_m2d, a_d2m, u_emb, y128, scal, *weights)


def kernel(miRNA_embedding_feature, miRNA_similarity_feature,
           miRNA_association_feature, disease_similarity_feature,
           disease_association_feature, edge_index_m2d, edge_index_d2m,
           params):
    p = params

    pad = E_PAD - E

    def prep(ei):
        srcp = jnp.concatenate([ei[0], jnp.zeros((pad,), jnp.int32)])
        dstp = jnp.concatenate([ei[1], jnp.full((pad,), N_D, jnp.int32)])
        return srcp, dstp

    src0, dst0 = prep(edge_index_m2d)
    src1, dst1 = prep(edge_index_d2m)

    u_emb, y128 = _emb_proj(miRNA_embedding_feature,
                            p["upd_emb"][0], p["upd_emb"][1],
                            p["emb_lin"][0], p["emb_lin"][1])

    a = _build_adj(src0, dst0, src1, dst1)
    a_m2d = a[:A_N].reshape(N_D, N_M)
    a_d2m = a[A_N:].reshape(N_M, N_D)

    scal = jnp.concatenate([
        p["w_m"], p["w_d"], p["w_m_u"], p["w_d_u"],
        p["agg_m"], p["agg_d"],
    ])

    def b2(lp):
        return lp

    weights = []
    for name in ("msim_lin", "mass_lin", "dsim_lin", "dass_lin",
                 "upd_msim", "upd_mass", "upd_dsim", "upd_dass",
                 "emb_lin_u", "msim_lin_u", "mass_lin_u",
                 "dsim_lin_u", "dass_lin_u"):
        weights.extend(b2(p[name]))
    for l in range(3):
        weights.extend(b2(p["sage_d2m_%d_l" % l]))
        weights.extend(b2(p["sage_d2m_%d_r" % l]))
    for l in range(3):
        weights.extend(b2(p["sage_m2d_%d_l" % l]))
        weights.extend(b2(p["sage_m2d_%d_r" % l]))

    out_m, out_d = _dense(miRNA_similarity_feature, miRNA_association_feature,
                          disease_similarity_feature,
                          disease_association_feature,
                          a_m2d, a_d2m, u_emb, y128, scal, weights)
    return out_m, out_d


# speedup vs baseline: 1.0036x; 1.0036x over previous
import functools

import jax
import jax.numpy as jnp
from jax import lax
from jax.experimental import pallas as pl
from jax.experimental.pallas import tpu as pltpu
from jax.experimental.pallas import tpu_sc as plsc

N_M = 800
N_D = 800
E = 50000
SLOPE = 0.2

NC = 2
NS = 16
CHUNK = 3136
E_PAD = NS * CHUNK
SUB = 112
NSUB = CHUNK // SUB
A_N = N_M * N_D
A_SIZE = 641024
ZCHUNK = A_SIZE // NS
OCHUNK = A_N // NS


def _leaky(x):
    return jnp.where(x >= 0, x, SLOPE * x)


def _adj_body(src0_hbm, dst0_hbm, src1_hbm, dst1_hbm, out_hbm,
              src_v, dst_v, idx_v, ones_v, stripe_v, a_sh, sem):
    s = lax.axis_index("s")

    def zbody(i, _):
        for u in range(8):
            stripe_v[pl.ds((i * 8 + u) * 16, 16)] = jnp.zeros((16,), jnp.float32)
        return 0
    lax.fori_loop(0, ZCHUNK // 128, zbody, 0)
    zdma0 = pltpu.async_copy(stripe_v, a_sh.at[pl.ds(s * ZCHUNK, ZCHUNK)], sem)
    zdma1 = pltpu.async_copy(
        stripe_v, a_sh.at[pl.ds(A_SIZE + s * ZCHUNK, ZCHUNK)], sem)

    base = s * CHUNK
    for u in range(SUB // 16):
        ones_v[pl.ds(u * 16, 16)] = jnp.ones((16,), jnp.float32)
    zdma0.wait()
    zdma1.wait()
    plsc.subcore_barrier()

    descs = []
    for et, (src_hbm, dst_hbm) in enumerate(((src0_hbm, dst0_hbm),
                                             (src1_hbm, dst1_hbm))):
        pltpu.sync_copy(src_hbm.at[pl.ds(base, CHUNK)], src_v)
        pltpu.sync_copy(dst_hbm.at[pl.ds(base, CHUNK)], dst_v)
        for t in range(NSUB):
            for u in range(SUB // 16):
                off = t * SUB + u * 16
                sv = src_v[pl.ds(off, 16)]
                dv = dst_v[pl.ds(off, 16)]
                idx_v[t, pl.ds(u * 16, 16)] = dv * N_D + sv + et * A_SIZE
            descs.append(pltpu.async_copy(ones_v, a_sh.at[idx_v.at[t]], sem,
                                          add=True))
        for d in descs:
            d.wait()
        descs = []

    plsc.subcore_barrier()

    for et in range(2):
        pltpu.sync_copy(a_sh.at[pl.ds(et * A_SIZE + s * OCHUNK, OCHUNK)],
                        stripe_v.at[pl.ds(0, OCHUNK)])
        pltpu.sync_copy(stripe_v.at[pl.ds(0, OCHUNK)],
                        out_hbm.at[pl.ds(et * A_N + s * OCHUNK, OCHUNK)])


def _build_adj(src0, dst0, src1, dst1):
    mesh = plsc.VectorSubcoreMesh(core_axis_name="c", subcore_axis_name="s",
                                  num_cores=1)
    fn = pl.kernel(
        _adj_body,
        out_type=jax.ShapeDtypeStruct((2 * A_N,), jnp.float32),
        mesh=mesh,
        scratch_types=[
            pltpu.VMEM((CHUNK,), jnp.int32),
            pltpu.VMEM((CHUNK,), jnp.int32),
            pltpu.VMEM((NSUB, SUB), jnp.int32),
            pltpu.VMEM((SUB,), jnp.float32),
            pltpu.VMEM((ZCHUNK,), jnp.float32),
            pltpu.VMEM_SHARED((2 * A_SIZE,), jnp.float32),
            pltpu.SemaphoreType.DMA,
        ],
    )
    return fn(src0, dst0, src1, dst1)


KB = 1920
KSTEPS = 96000 // KB


def _emb_body(emb_ref, wu_ref, we_ref, bu_ref, be_ref, u_out, y_out,
              acc_u, acc_y):
    k = pl.program_id(0)

    @pl.when(k == 0)
    def _():
        acc_u[...] = jnp.zeros_like(acc_u)
        acc_y[...] = jnp.zeros_like(acc_y)

    e = emb_ref[...].astype(jnp.bfloat16)
    dn = (((1,), (1,)), ((), ()))
    acc_u[...] += lax.dot_general(e, wu_ref[...].astype(jnp.bfloat16), dn,
                                  preferred_element_type=jnp.float32)
    acc_y[...] += lax.dot_general(e, we_ref[...].astype(jnp.bfloat16), dn,
                                  preferred_element_type=jnp.float32)

    @pl.when(k == KSTEPS - 1)
    def _():
        u_out[...] = _leaky(acc_u[...] + bu_ref[...][None, :]).astype(jnp.bfloat16)
        y_out[...] = acc_y[...] + be_ref[...][None, :]


def _emb_proj(emb, wu, bu, we, be):
    return pl.pallas_call(
        _emb_body,
        grid=(KSTEPS,),
        in_specs=[
            pl.BlockSpec((N_M, KB), lambda k: (0, k)),
            pl.BlockSpec((1024, KB), lambda k: (0, k)),
            pl.BlockSpec((128, KB), lambda k: (0, k)),
            pl.BlockSpec((1024,), lambda k: (0,)),
            pl.BlockSpec((128,), lambda k: (0,)),
        ],
        out_specs=[
            pl.BlockSpec((N_M, 1024), lambda k: (0, 0)),
            pl.BlockSpec((N_M, 128), lambda k: (0, 0)),
        ],
        out_shape=[
            jax.ShapeDtypeStruct((N_M, 1024), jnp.bfloat16),
            jax.ShapeDtypeStruct((N_M, 128), jnp.float32),
        ],
        scratch_shapes=[
            pltpu.VMEM((N_M, 1024), jnp.float32),
            pltpu.VMEM((N_M, 128), jnp.float32),
        ],
    )(emb, wu, we, bu, be)


def _dense_body(msim, mass, dsim, dass, a_m2d, a_d2m, u_emb, y128, scal,
                w_msim_l, b_msim_l, w_mass_l, b_mass_l,
                w_dsim_l, b_dsim_l, w_dass_l, b_dass_l,
                w_umsim, b_umsim, w_umass, b_umass,
                w_udsim, b_udsim, w_udass, b_udass,
                w_emb_u, b_emb_u, w_msim_u, b_msim_u, w_mass_u, b_mass_u,
                w_dsim_u, b_dsim_u, w_dass_u, b_dass_u,
                s0l_w, s0l_b, s0r_w, s0r_b, s1l_w, s1l_b, s1r_w, s1r_b,
                s2l_w, s2l_b, s2r_w, s2r_b,
                t0l_w, t0l_b, t0r_w, t0r_b, t1l_w, t1l_b, t1r_w, t1r_b,
                t2l_w, t2l_b, t2r_w, t2r_b,
                out_m, out_d,
                msim_v, mass_v, dsim_v, dass_v, wum_v, wua_v, wuds_v, wuda_v,
                uemb_v, y128_v, am_v, ad_v, sems):
    dn = (((1,), (1,)), ((), ()))

    def mm(x, w):
        return lax.dot_general(x.astype(jnp.bfloat16), w[...].astype(jnp.bfloat16),
                               dn, preferred_element_type=jnp.float32)

    def app(x, w, b):
        return mm(x, w) + b[...][None, :]

    big = [(y128, y128_v), (msim, msim_v), (mass, mass_v),
           (dsim, dsim_v), (dass, dass_v),
           (w_umsim, wum_v), (w_umass, wua_v),
           (w_udsim, wuds_v), (w_udass, wuda_v),
           (u_emb, uemb_v),
           (a_d2m, am_v), (a_m2d, ad_v)]
    descs = [pltpu.async_copy(s, d, sems.at[i]) for i, (s, d) in enumerate(big)]

    e = jnp.exp(scal[...])
    def sm(base, n, i):
        tot = e[base]
        for j in range(1, n):
            tot = tot + e[base + j]
        return e[base + i] / tot
    w_m0, w_m1, w_m2 = sm(0, 3, 0), sm(0, 3, 1), sm(0, 3, 2)
    w_d0, w_d1 = sm(3, 2, 0), sm(3, 2, 1)
    w_mu0, w_mu1, w_mu2 = sm(5, 3, 0), sm(5, 3, 1), sm(5, 3, 2)
    w_du0, w_du1 = sm(8, 2, 0), sm(8, 2, 1)
    ag_m0, ag_m1 = sm(10, 2, 0), sm(10, 2, 1)
    ag_d0, ag_d1 = sm(12, 2, 0), sm(12, 2, 1)

    descs[0].wait()
    descs[1].wait()
    descs[2].wait()
    ms = msim_v[...]
    ma = mass_v[...]
    x_m = y128_v[...] * w_m0 + app(ms, w_msim_l, b_msim_l) * w_m1 \
        + app(ma, w_mass_l, b_mass_l) * w_m2

    descs[3].wait()
    descs[4].wait()
    ds_ = dsim_v[...]
    da = dass_v[...]
    x_d = app(ds_, w_dsim_l, b_dsim_l) * w_d0 + app(da, w_dass_l, b_dass_l) * w_d1

    descs[5].wait()
    u_msim = _leaky(app(ms, wum_v, b_umsim))
    descs[6].wait()
    u_mass = _leaky(app(ma, wua_v, b_umass))
    descs[7].wait()
    u_dsim = _leaky(app(ds_, wuds_v, b_udsim))
    descs[8].wait()
    u_dass = _leaky(app(da, wuda_v, b_udass))

    descs[9].wait()
    xm = app(uemb_v[...], w_emb_u, b_emb_u) * w_mu0 \
        + app(u_msim, w_msim_u, b_msim_u) * w_mu1 \
        + app(u_mass, w_mass_u, b_mass_u) * w_mu2
    xd = app(u_dsim, w_dsim_u, b_dsim_u) * w_du0 \
        + app(u_dass, w_dass_u, b_dass_u) * w_du1

    descs[10].wait()
    descs[11].wait()
    am = am_v[...]
    ad = ad_v[...]
    inv_m = 1.0 / jnp.maximum(jnp.sum(am, axis=1, keepdims=True), 1.0)
    inv_d = 1.0 / jnp.maximum(jnp.sum(ad, axis=1, keepdims=True), 1.0)

    def mean_m(x):
        return jnp.dot(am.astype(jnp.bfloat16), x.astype(jnp.bfloat16),
                       preferred_element_type=jnp.float32) * inv_m

    def mean_d(x):
        return jnp.dot(ad.astype(jnp.bfloat16), x.astype(jnp.bfloat16),
                       preferred_element_type=jnp.float32) * inv_d

    nm = app(mean_m(xd), s0l_w, s0l_b) + app(xm, s0r_w, s0r_b)
    nd = app(mean_d(xm), t0l_w, t0l_b) + app(xd, t0r_w, t0r_b)
    xm, xd = _leaky(nm), _leaky(nd)
    nm = app(mean_m(xd), s1l_w, s1l_b) + app(xm, s1r_w, s1r_b)
    nd = app(mean_d(xm), t1l_w, t1l_b) + app(xd, t1r_w, t1r_b)
    xm, xd = _leaky(nm), _leaky(nd)
    nm = app(mean_m(xd), s2l_w, s2l_b) + app(xm, s2r_w, s2r_b)
    nd = app(mean_d(xm), t2l_w, t2l_b) + app(xd, t2r_w, t2r_b)

    out_m[...] = x_m * ag_m0 + nm * ag_m1
    out_d[...] = x_d * ag_d0 + nd * ag_d1


_BIG_W_IDX = {8, 10, 12, 14}


def _dense(msim, mass, dsim, dass, a_m2d, a_d2m, u_emb, y128, scal, weights):
    in_specs = ([pl.BlockSpec(memory_space=pl.ANY)] * 8
                + [pl.BlockSpec((14,), lambda: (0,))]
                + [pl.BlockSpec(memory_space=pl.ANY) if i in _BIG_W_IDX
                   else pl.BlockSpec(w.shape, lambda nd=w.ndim: (0,) * nd)
                   for i, w in enumerate(weights)])
    return pl.pallas_call(
        _dense_body,
        in_specs=in_specs,
        out_specs=[
            pl.BlockSpec((N_M, 128), lambda: (0, 0)),
            pl.BlockSpec((N_D, 128), lambda: (0, 0)),
        ],
        out_shape=[
            jax.ShapeDtypeStruct((N_M, 128), jnp.float32),
            jax.ShapeDtypeStruct((N_D, 128), jnp.float32),
        ],
        scratch_shapes=[
            pltpu.VMEM((N_M, N_M), jnp.float32),
            pltpu.VMEM((N_M, N_D), jnp.float32),
            pltpu.VMEM((N_D, N_D), jnp.float32),
            pltpu.VMEM((N_D, N_M), jnp.float32),
            pltpu.VMEM((N_M, N_M), jnp.float32),
            pltpu.VMEM((N_D, N_D), jnp.float32),
            pltpu.VMEM((N_D, N_D), jnp.float32),
            pltpu.VMEM((N_M, N_M), jnp.float32),
            pltpu.VMEM((N_M, 1024), jnp.bfloat16),
            pltpu.VMEM((N_M, 128), jnp.float32),
            pltpu.VMEM((N_M, N_D), jnp.float32),
            pltpu.VMEM((N_D, N_M), jnp.float32),
            pltpu.SemaphoreType.DMA((12,)),
        ],
    )(msim, mass, dsim, dass, a_m2d, a_d2m, u_emb, y128, scal, *weights)


def kernel(miRNA_embedding_feature, miRNA_similarity_feature,
           miRNA_association_feature, disease_similarity_feature,
           disease_association_feature, edge_index_m2d, edge_index_d2m,
           params):
    p = params

    pad = E_PAD - E

    def prep(ei):
        srcp = jnp.concatenate([ei[0], jnp.zeros((pad,), jnp.int32)])
        dstp = jnp.concatenate([ei[1], jnp.full((pad,), N_D, jnp.int32)])
        return srcp, dstp

    src0, dst0 = prep(edge_index_m2d)
    src1, dst1 = prep(edge_index_d2m)

    u_emb, y128 = _emb_proj(miRNA_embedding_feature,
                            p["upd_emb"][0], p["upd_emb"][1],
                            p["emb_lin"][0], p["emb_lin"][1])

    a = _build_adj(src0, dst0, src1, dst1)
    a_m2d = a[:A_N].reshape(N_D, N_M)
    a_d2m = a[A_N:].reshape(N_M, N_D)

    scal = jnp.concatenate([
        p["w_m"], p["w_d"], p["w_m_u"], p["w_d_u"],
        p["agg_m"], p["agg_d"],
    ])

    def b2(lp):
        return lp

    weights = []
    for name in ("msim_lin", "mass_lin", "dsim_lin", "dass_lin",
                 "upd_msim", "upd_mass", "upd_dsim", "upd_dass",
                 "emb_lin_u", "msim_lin_u", "mass_lin_u",
                 "dsim_lin_u", "dass_lin_u"):
        weights.extend(b2(p[name]))
    for l in range(3):
        weights.extend(b2(p["sage_d2m_%d_l" % l]))
        weights.extend(b2(p["sage_d2m_%d_r" % l]))
    for l in range(3):
        weights.extend(b2(p["sage_m2d_%d_l" % l]))
        weights.extend(b2(p["sage_m2d_%d_r" % l]))

    out_m, out_d = _dense(miRNA_similarity_feature, miRNA_association_feature,
                          disease_similarity_feature,
                          disease_association_feature,
                          a_m2d, a_d2m, u_emb, y128, scal, weights)
    return out_m, out_d
